# Initial kernel scaffold; baseline (speedup 1.0000x reference)
import jax
import jax.numpy as jnp
from jax.experimental import pallas as pl


def kernel(t_cls_scores, t_bbox_preds, t_centernesses):
    raise NotImplementedError("write your pallas kernel here")


# speedup vs baseline: 2.5801x; 2.5801x over previous
import functools

import jax
import jax.numpy as jnp
from jax import lax
from jax.experimental import pallas as pl
from jax.experimental.pallas import tpu as pltpu
from jax.experimental.pallas import tpu_sc as plsc

N = 174592
K = 1745
NT = 16
C = N // NT
NV = C // 16
PASSES = ((22, 256), (14, 256), (7, 128), (0, 128))
HW = 4096
AW = HW + 16

DR = 2728
DCW = 1024
DCS = 64
BR = 88
GB = 31


def _dense_body(cls_ref, cen_ref, p_ref, sc_ref, w_ref, sum_ref):
    i = pl.program_id(0)
    x = cls_ref[...]
    m = jnp.maximum(x, pltpu.roll(x, 1, 1))
    m = jnp.maximum(m, pltpu.roll(m, 2, 1))
    m = jnp.maximum(m, pltpu.roll(m, 4, 1))
    m = jnp.maximum(m, pltpu.roll(m, 8, 1))
    mc = jax.lax.dot_general(
        m, p_ref[...], (((1,), (0,)), ((), ())),
        precision=jax.lax.Precision.HIGHEST,
        preferred_element_type=jnp.float32,
    )
    s = jax.nn.sigmoid(mc)
    w = s * jax.nn.sigmoid(cen_ref[...])
    sc_ref[...] = s
    w_ref[...] = w
    blk = jnp.sum(s)
    prev = jnp.where(i == 0, 0.0, sum_ref[0, 0])
    acc = prev + blk
    sum_ref[...] = jnp.where(i == GB - 1, acc / N, acc).reshape(1, 1)


def _dense(cls2, cen2, p):
    return pl.pallas_call(
        _dense_body,
        grid=(GB,),
        in_specs=[
            pl.BlockSpec((BR, DCW), lambda i: (i, 0)),
            pl.BlockSpec((BR, DCS), lambda i: (i, 0)),
            pl.BlockSpec((DCW, DCS), lambda i: (0, 0)),
        ],
        out_specs=[
            pl.BlockSpec((BR, DCS), lambda i: (i, 0)),
            pl.BlockSpec((BR, DCS), lambda i: (i, 0)),
            pl.BlockSpec((1, 1), lambda i: (0, 0)),
        ],
        out_shape=[
            jax.ShapeDtypeStruct((DR, DCS), jnp.float32),
            jax.ShapeDtypeStruct((DR, DCS), jnp.float32),
            jax.ShapeDtypeStruct((1, 1), jnp.float32),
        ],
    )(cls2, cen2, p)


def _lane():
    return lax.iota(jnp.int32, 16)


def _lane0(vec):
    return jnp.sum(jnp.where(_lane() == 0, vec, jnp.zeros_like(vec)))


def _bcast(x):
    return jnp.zeros((16,), jnp.int32) + x


def _sel_body(
    scores_hbm,
    mask_hbm,
    stats_hbm,
    scores_v,
    mask_v,
    hist_p,
    hist_n,
    tmp2,
    gh_v,
    a_v,
    gsl,
    ctrl_v,
    fsum_v,
    stats_v,
    sh_hist_p,
    sh_hist_n,
    sh_gh_p,
    sh_gh_n,
    sh_ctrl,
    sh_cnt,
    sh_sum,
):
    tid = lax.axis_index("s")
    lane = _lane()
    base = tid * C
    pltpu.sync_copy(scores_hbm.at[pl.ds(base, C)], scores_v)

    kpos = jnp.int32(K)
    kneg = jnp.int32(K)
    ppos = jnp.int32(0)
    pneg = jnp.int32(0)

    for (sh, B) in PASSES:
        W = B.bit_length() - 1
        HI = sh + W
        BW = B * 16
        SW = BW // NT

        def zbody(j, _):
            z = jnp.zeros((16,), jnp.int32)
            hist_p[pl.ds(j * 16, 16)] = z
            hist_n[pl.ds(j * 16, 16)] = z
            return 0

        lax.fori_loop(0, B, zbody, 0)

        ppos_v = _bcast(ppos)
        pneg_v = _bcast(pneg)
        one = jnp.ones((16,), jnp.int32)

        def sbody(i, _):
            v = scores_v[pl.ds(i * 16, 16)]
            u = plsc.bitcast(v, jnp.int32)
            d = jnp.right_shift(u, sh) & (B - 1)
            hi = jnp.right_shift(u, HI)
            idx = d * 16 + lane
            plsc.addupdate_scatter(hist_p, [idx], one, mask=hi == ppos_v)
            plsc.addupdate_scatter(hist_n, [idx], one, mask=hi == pneg_v)
            return 0

        lax.fori_loop(0, NV, sbody, 0)

        pltpu.sync_copy(hist_p.at[pl.ds(0, BW)], sh_hist_p.at[tid, pl.ds(0, BW)])
        pltpu.sync_copy(hist_n.at[pl.ds(0, BW)], sh_hist_n.at[tid, pl.ds(0, BW)])
        plsc.subcore_barrier()

        for sh_hist, sh_gh in ((sh_hist_p, sh_gh_p), (sh_hist_n, sh_gh_n)):
            for r in range(NT):
                pltpu.sync_copy(
                    sh_hist.at[r, pl.ds(tid * SW, SW)], tmp2.at[pl.ds(r * SW, SW)]
                )

            def jbody(j, _):
                acc = jnp.zeros((16,), jnp.int32)
                for r in range(NT):
                    acc = acc + tmp2[pl.ds(r * SW + j * 16, 16)]
                gsl[pl.ds(j * 16, 16)] = acc
                return 0

            lax.fori_loop(0, SW // 16, jbody, 0)
            pltpu.sync_copy(gsl.at[pl.ds(0, SW)], sh_gh.at[pl.ds(tid * SW, SW)])
        plsc.subcore_barrier()

        @pl.when(tid == 0)
        def _():
            def prefix_and_total(_):
                def pbody(j, carry):
                    g = gh_v[pl.ds(j * 16, 16)]
                    cs = plsc.cumsum(g)
                    a_v[pl.ds(j * 16, 16)] = cs - g + carry
                    return carry + jnp.sum(g)

                total = lax.fori_loop(0, BW // 16, pbody, jnp.int32(0))
                a_v[pl.ds(BW, 16)] = _bcast(total)
                return total

            pltpu.sync_copy(sh_gh_p.at[pl.ds(0, BW)], gh_v.at[pl.ds(0, BW)])
            total_p = prefix_and_total(None)
            target = total_p - kpos

            def cbody(j, cnt):
                bvec = j * 16 + lane
                e = plsc.load_gather(a_v, [bvec * 16])
                return cnt + jnp.sum(jnp.where(e <= target, 1, 0))

            bstar = lax.fori_loop(0, B // 16, cbody, jnp.int32(0)) - 1
            eb1 = jnp.max(plsc.load_gather(a_v, [_bcast((bstar + 1) * 16)]))
            kpos_n = kpos - (total_p - eb1)
            ppos_n = ppos * B + bstar

            pltpu.sync_copy(sh_gh_n.at[pl.ds(0, BW)], gh_v.at[pl.ds(0, BW)])
            prefix_and_total(None)

            def nbody(j, cnt):
                bvec = j * 16 + lane
                f = plsc.load_gather(a_v, [(bvec + 1) * 16])
                return cnt + jnp.sum(jnp.where(f < kneg, 1, 0))

            bn = lax.fori_loop(0, B // 16, nbody, jnp.int32(0))
            ebn = jnp.max(plsc.load_gather(a_v, [_bcast(bn * 16)]))
            kneg_n = kneg - ebn
            pneg_n = pneg * B + bn

            ctrl = (
                jnp.where(lane == 0, ppos_n, 0)
                + jnp.where(lane == 1, kpos_n, 0)
                + jnp.where(lane == 2, pneg_n, 0)
                + jnp.where(lane == 3, kneg_n, 0)
            )
            ctrl_v[...] = ctrl
            pltpu.sync_copy(ctrl_v, sh_ctrl)

        plsc.subcore_barrier()
        pltpu.sync_copy(sh_ctrl, ctrl_v)
        cv = ctrl_v[...]
        ppos = jnp.sum(jnp.where(lane == 0, cv, 0))
        kpos = jnp.sum(jnp.where(lane == 1, cv, 0))
        pneg = jnp.sum(jnp.where(lane == 2, cv, 0))
        kneg = jnp.sum(jnp.where(lane == 3, cv, 0))

    tpos_v = _bcast(ppos)
    tneg_v = _bcast(pneg)

    def stbody(i, carry):
        ep, en, sg = carry
        v = scores_v[pl.ds(i * 16, 16)]
        u = plsc.bitcast(v, jnp.int32)
        ep = ep + jnp.sum(jnp.where(u == tpos_v, 1, 0))
        en = en + jnp.sum(jnp.where(u == tneg_v, 1, 0))
        sg = sg + jnp.sum(jnp.where(u > tpos_v, v, 0.0))
        return ep, en, sg

    ep, en, sg = lax.fori_loop(
        0, NV, stbody, (jnp.int32(0), jnp.int32(0), jnp.float32(0.0))
    )
    ctrl_v[...] = jnp.where(lane == 0, ep, 0) + jnp.where(lane == 1, en, 0)
    pltpu.sync_copy(ctrl_v, sh_cnt.at[pl.ds(tid * 16, 16)])
    stats_v[...] = jnp.where(lane == 0, sg, 0.0)
    pltpu.sync_copy(stats_v, sh_sum.at[pl.ds(tid * 16, 16)])
    plsc.subcore_barrier()

    pltpu.sync_copy(sh_cnt, tmp2.at[pl.ds(0, 256)])
    pltpu.sync_copy(sh_sum, fsum_v)
    eqp_all = plsc.load_gather(tmp2, [lane * 16])
    eqn_all = plsc.load_gather(tmp2, [lane * 16 + 1])
    sg_all = plsc.load_gather(fsum_v, [lane * 16])
    pre_p = jnp.sum(jnp.where(lane < tid, eqp_all, 0))
    pre_n = jnp.sum(jnp.where(lane < tid, eqn_all, 0))
    sg_tot = jnp.sum(sg_all)

    @pl.when(tid == 0)
    def _():
        tpos_f = jnp.max(plsc.bitcast(tpos_v, jnp.float32))
        fg = sg_tot + kpos.astype(jnp.float32) * tpos_f
        stats_v[...] = jnp.where(lane == 0, fg, 0.0)
        pltpu.sync_copy(stats_v, stats_hbm)

    def mbody(i, carry):
        cp, cn = carry
        v = scores_v[pl.ds(i * 16, 16)]
        u = plsc.bitcast(v, jnp.int32)
        eqp = u == tpos_v
        ei = jnp.where(eqp, 1, 0)
        csp = plsc.cumsum(ei)
        takep = eqp & (cp + csp - 1 < kpos)
        eqn = u == tneg_v
        ni = jnp.where(eqn, 1, 0)
        csn = plsc.cumsum(ni)
        taken = eqn & (cn + csn - 1 < kneg)
        m = jnp.where(
            (u < tneg_v) | taken,
            -1.0,
            jnp.where((u > tpos_v) | takep, 1.0, 0.0),
        )
        mask_v[pl.ds(i * 16, 16)] = m
        return cp + jnp.sum(ei), cn + jnp.sum(ni)

    lax.fori_loop(0, NV, mbody, (pre_p, pre_n))
    pltpu.sync_copy(mask_v, mask_hbm.at[pl.ds(base, C)])


@functools.cache
def _get_select():
    return pl.kernel(
        _sel_body,
        out_type=[
            jax.ShapeDtypeStruct((N,), jnp.float32),
            jax.ShapeDtypeStruct((16,), jnp.float32),
        ],
        mesh=plsc.VectorSubcoreMesh(
            core_axis_name="c", subcore_axis_name="s", num_cores=1
        ),
        compiler_params=pltpu.CompilerParams(needs_layout_passes=False),
    scratch_types=[
        pltpu.VMEM((C,), jnp.float32),
        pltpu.VMEM((C,), jnp.float32),
        pltpu.VMEM((HW,), jnp.int32),
        pltpu.VMEM((HW,), jnp.int32),
        pltpu.VMEM((HW,), jnp.int32),
        pltpu.VMEM((HW,), jnp.int32),
        pltpu.VMEM((AW,), jnp.int32),
        pltpu.VMEM((256,), jnp.int32),
        pltpu.VMEM((16,), jnp.int32),
        pltpu.VMEM((256,), jnp.float32),
        pltpu.VMEM((16,), jnp.float32),
        pltpu.VMEM_SHARED((16, HW), jnp.int32),
        pltpu.VMEM_SHARED((16, HW), jnp.int32),
        pltpu.VMEM_SHARED((HW,), jnp.int32),
        pltpu.VMEM_SHARED((HW,), jnp.int32),
        pltpu.VMEM_SHARED((16,), jnp.int32),
        pltpu.VMEM_SHARED((256,), jnp.int32),
        pltpu.VMEM_SHARED((256,), jnp.float32),
        ],
    )


def kernel(t_cls_scores, t_bbox_preds, t_centernesses):
    cls2 = t_cls_scores.reshape(DR, DCW)
    cen2 = t_centernesses.reshape(DR, DCS)
    p = jnp.equal(
        lax.broadcasted_iota(jnp.int32, (DCW, DCS), 0),
        lax.broadcasted_iota(jnp.int32, (DCW, DCS), 1) * 16 + 15,
    ).astype(jnp.float32)
    scores2, weight2, sdps = _dense(cls2, cen2, p)
    mask, stats = _get_select()(scores2.reshape(N))
    pos_mask = mask > 0.0
    neg_mask = mask < 0.0
    return pos_mask, neg_mask, weight2.reshape(N), stats[0], sdps[0, 0]
